# baseline (speedup 1.0000x reference)
import functools

import jax
import jax.numpy as jnp
from jax import lax
from jax.experimental import pallas as pl
from jax.experimental.pallas import tpu as pltpu
from jax.experimental.pallas import tpu_sc as plsc

_N = 16
_H = 512
_W = 512
_L = _H * _W
_M = int(0.1 * _L)

_NSC = 2
_NSUB = 16
_NW = _NSC * _NSUB
_HALF = _L * _N // _NW
_CHUNK = 16384
_NCHUNK = _HALF // _CHUNK
_BINS = 8192
_UNROLL = 8


def _dense_body(lg_ref, tg_ref, stats_ref):
    lg = lg_ref[0]
    tg = tg_ref[0]
    m1 = jax.nn.sigmoid(lg)
    s1 = jnp.sum(m1)
    s2 = jnp.sum(tg)
    s12 = jnp.sum(m1 * tg)
    idx = lax.broadcasted_iota(jnp.int32, (1, 4), 1)
    stats_ref[0] = jnp.where(
        idx == 0, s1, jnp.where(idx == 1, s2, jnp.where(idx == 2, s12, 0.0))
    )


_ROWS = _CHUNK // _W


def _sc_hist_body(lg_hbm, tg_hbm, out_hbm, lchunk_v, tchunk_v, sum_v):
    wid = lax.axis_index("s") * _NSC + lax.axis_index("c")
    smp = wid // 2
    row0 = (wid % 2) * (_H // 2)
    zeros = jnp.zeros((16,), jnp.float32)

    def zbody(i, carry):
        sum_v[pl.ds(i * 16, 16)] = zeros
        return carry

    lax.fori_loop(0, _BINS // 16, zbody, 0)

    def cbody(c, carry):
        rows = pl.ds(row0 + c * _ROWS, _ROWS)
        pltpu.sync_copy(lg_hbm.at[smp, rows], lchunk_v)
        pltpu.sync_copy(tg_hbm.at[smp, rows], tchunk_v)

        def ibody(r, icarry):
            nslc = _W // 16
            vs = [
                (1.0 - tchunk_v[r, pl.ds(j * 16, 16)])
                / (1.0 + jnp.exp(lchunk_v[r, pl.ds(j * 16, 16)]))
                for j in range(nslc)
            ]
            bs = [
                jnp.minimum(
                    lax.convert_element_type(v * float(_BINS), jnp.int32),
                    _BINS - 1,
                )
                for v in vs
            ]
            for v, b in zip(vs, bs):
                plsc.addupdate_scatter(sum_v, [b], v)
            return icarry

        lax.fori_loop(0, _ROWS, ibody, 0)
        return carry

    lax.fori_loop(0, _NCHUNK, cbody, 0)
    pltpu.sync_copy(sum_v, out_hbm.at[wid])


@functools.cache
def _make_sc_hist():
    mesh = plsc.VectorSubcoreMesh(
        core_axis_name="c",
        subcore_axis_name="s",
        num_cores=_NSC,
        num_subcores=_NSUB,
    )
    return pl.kernel(
        _sc_hist_body,
        out_type=jax.ShapeDtypeStruct((_NW, _BINS), jnp.float32),
        mesh=mesh,
        scratch_types=[
            pltpu.VMEM((_ROWS, _W), jnp.float32),
            pltpu.VMEM((_ROWS, _W), jnp.float32),
            pltpu.VMEM((_BINS,), jnp.float32),
        ],
        compiler_params=pltpu.CompilerParams(needs_layout_passes=False),
    )


def _fin_body(hist_ref, topm_ref):
    h = hist_ref[...].reshape(_N, 2, _BINS)
    hsum = h[:, 0] + h[:, 1]
    iota_b = lax.broadcasted_iota(jnp.int32, (_N, _BINS), 1)
    cnt_lo = hsum * (
        float(_BINS) / (lax.convert_element_type(iota_b, jnp.float32) + 1.0)
    )
    rc = cnt_lo
    k = 1
    while k < _BINS:
        rc = rc + jnp.concatenate(
            [rc[:, k:], jnp.zeros((_N, k), jnp.float32)], axis=1
        )
        k *= 2
    bstar = jnp.max(
        jnp.where(rc >= float(_M), iota_b, 0), axis=1, keepdims=True
    )
    t = lax.convert_element_type(bstar, jnp.float32) * (1.0 / float(_BINS))
    above = iota_b >= bstar
    c_above = jnp.sum(jnp.where(above, cnt_lo, 0.0), axis=1, keepdims=True)
    s_above = jnp.sum(jnp.where(above, hsum, 0.0), axis=1, keepdims=True)
    topm_ref[...] = s_above - t * c_above + float(_M) * t


def kernel(logits, targets):
    stats = pl.pallas_call(
        _dense_body,
        grid=(_N,),
        in_specs=[
            pl.BlockSpec((1, _H, _W), lambda i: (i, 0, 0)),
            pl.BlockSpec((1, _H, _W), lambda i: (i, 0, 0)),
        ],
        out_specs=pl.BlockSpec((1, 1, 4), lambda i: (i, 0, 0)),
        out_shape=jax.ShapeDtypeStruct((_N, 1, 4), jnp.float32),
    )(logits, targets)

    hists = _make_sc_hist()(logits, targets)

    topm = pl.pallas_call(
        _fin_body,
        out_shape=jax.ShapeDtypeStruct((_N, 1), jnp.float32),
    )(hists)

    s1 = stats[:, 0, 0]
    s2 = stats[:, 0, 1]
    s12 = stats[:, 0, 2]
    tm = topm[:, 0]
    score = 2.0 * (s12 + 1.0) / (s1 + 2.0 * s2 - s12 + tm + 1.0)
    return (1.0 - jnp.sum(score) / _N).astype(jnp.float32)


# speedup vs baseline: 1.5761x; 1.5761x over previous
import functools

import jax
import jax.numpy as jnp
from jax import lax
from jax.experimental import pallas as pl
from jax.experimental.pallas import tpu as pltpu
from jax.experimental.pallas import tpu_sc as plsc

_N = 16
_H = 512
_W = 512
_L = _H * _W
_M = int(0.1 * _L)

_NSC = 2
_NSUB = 16
_NW = _NSC * _NSUB
_HALF = _L * _N // _NW
_CHUNK = 16384
_NCHUNK = _HALF // _CHUNK
_BINS = 8192
_UNROLL = 8


def _dense_body(lg_ref, tg_ref, stats_ref, tn_ref):
    lg = lg_ref[0]
    tg = tg_ref[0]
    m1 = jax.nn.sigmoid(lg)
    tn = (1.0 - m1) * (1.0 - tg)
    tn_ref[0] = tn
    s1 = jnp.sum(m1)
    s2 = jnp.sum(tg)
    s12 = jnp.sum(m1 * tg)
    idx = lax.broadcasted_iota(jnp.int32, (1, 4), 1)
    stats_ref[0] = jnp.where(
        idx == 0, s1, jnp.where(idx == 1, s2, jnp.where(idx == 2, s12, 0.0))
    )


_ROWS = _CHUNK // _W


def _sc_hist_body(tn_hbm, out_hbm, chunk_v, sum_v):
    wid = lax.axis_index("s") * _NSC + lax.axis_index("c")
    smp = wid // 2
    row0 = (wid % 2) * (_H // 2)
    zeros = jnp.zeros((16,), jnp.float32)

    def zbody(i, carry):
        sum_v[pl.ds(i * 16, 16)] = zeros
        return carry

    lax.fori_loop(0, _BINS // 16, zbody, 0)

    def cbody(c, carry):
        pltpu.sync_copy(
            tn_hbm.at[smp, pl.ds(row0 + c * _ROWS, _ROWS)], chunk_v
        )

        def ibody(r, icarry):
            vs = [chunk_v[r, pl.ds(j * 16, 16)] for j in range(_W // 16)]
            bs = [
                jnp.minimum(
                    lax.convert_element_type(v * float(_BINS), jnp.int32),
                    _BINS - 1,
                )
                for v in vs
            ]
            for v, b in zip(vs, bs):
                plsc.addupdate_scatter(sum_v, [b], v)
            return icarry

        lax.fori_loop(0, _ROWS, ibody, 0)
        return carry

    lax.fori_loop(0, _NCHUNK, cbody, 0)
    pltpu.sync_copy(sum_v, out_hbm.at[wid])


@functools.cache
def _make_sc_hist():
    mesh = plsc.VectorSubcoreMesh(
        core_axis_name="c",
        subcore_axis_name="s",
        num_cores=_NSC,
        num_subcores=_NSUB,
    )
    return pl.kernel(
        _sc_hist_body,
        out_type=jax.ShapeDtypeStruct((_NW, _BINS), jnp.float32),
        mesh=mesh,
        scratch_types=[
            pltpu.VMEM((_ROWS, _W), jnp.float32),
            pltpu.VMEM((_BINS,), jnp.float32),
        ],
        compiler_params=pltpu.CompilerParams(needs_layout_passes=False),
    )


def _fin_body(hist_ref, topm_ref):
    h = hist_ref[...].reshape(_N, 2, _BINS)
    hsum = h[:, 0] + h[:, 1]
    iota_b = lax.broadcasted_iota(jnp.int32, (_N, _BINS), 1)
    cnt_lo = hsum * (
        float(_BINS) / (lax.convert_element_type(iota_b, jnp.float32) + 1.0)
    )
    rc = cnt_lo
    k = 1
    while k < _BINS:
        rc = rc + jnp.concatenate(
            [rc[:, k:], jnp.zeros((_N, k), jnp.float32)], axis=1
        )
        k *= 2
    bstar = jnp.max(
        jnp.where(rc >= float(_M), iota_b, 0), axis=1, keepdims=True
    )
    t = lax.convert_element_type(bstar, jnp.float32) * (1.0 / float(_BINS))
    above = iota_b >= bstar
    c_above = jnp.sum(jnp.where(above, cnt_lo, 0.0), axis=1, keepdims=True)
    s_above = jnp.sum(jnp.where(above, hsum, 0.0), axis=1, keepdims=True)
    topm_ref[...] = s_above - t * c_above + float(_M) * t


def kernel(logits, targets):
    stats, tn = pl.pallas_call(
        _dense_body,
        grid=(_N,),
        in_specs=[
            pl.BlockSpec((1, _H, _W), lambda i: (i, 0, 0)),
            pl.BlockSpec((1, _H, _W), lambda i: (i, 0, 0)),
        ],
        out_specs=[
            pl.BlockSpec((1, 1, 4), lambda i: (i, 0, 0)),
            pl.BlockSpec((1, _H, _W), lambda i: (i, 0, 0)),
        ],
        out_shape=[
            jax.ShapeDtypeStruct((_N, 1, 4), jnp.float32),
            jax.ShapeDtypeStruct((_N, _H, _W), jnp.float32),
        ],
    )(logits, targets)

    hists = _make_sc_hist()(tn)

    topm = pl.pallas_call(
        _fin_body,
        out_shape=jax.ShapeDtypeStruct((_N, 1), jnp.float32),
    )(hists)

    s1 = stats[:, 0, 0]
    s2 = stats[:, 0, 1]
    s12 = stats[:, 0, 2]
    tm = topm[:, 0]
    score = 2.0 * (s12 + 1.0) / (s1 + 2.0 * s2 - s12 + tm + 1.0)
    return (1.0 - jnp.sum(score) / _N).astype(jnp.float32)


# speedup vs baseline: 1.8171x; 1.1529x over previous
import functools

import jax
import jax.numpy as jnp
from jax import lax
from jax.experimental import pallas as pl
from jax.experimental.pallas import tpu as pltpu
from jax.experimental.pallas import tpu_sc as plsc

_N = 16
_H = 512
_W = 512
_L = _H * _W
_M = int(0.1 * _L)

_NSC = 2
_NSUB = 16
_NW = _NSC * _NSUB
_HALF = _L * _N // _NW
_CHUNK = 16384
_NCHUNK = _HALF // _CHUNK
_BINS = 8192
_UNROLL = 8


def _dense_body(lg_ref, tg_ref, stats_ref, tn_ref):
    lg = lg_ref[0]
    tg = tg_ref[0]
    m1 = jax.nn.sigmoid(lg)
    tn = (1.0 - m1) * (1.0 - tg)
    tn_ref[0] = tn
    s1 = jnp.sum(m1)
    s2 = jnp.sum(tg)
    s12 = jnp.sum(m1 * tg)
    idx = lax.broadcasted_iota(jnp.int32, (1, 4), 1)
    stats_ref[0] = jnp.where(
        idx == 0, s1, jnp.where(idx == 1, s2, jnp.where(idx == 2, s12, 0.0))
    )


_ROWS = _CHUNK // _W


def _sc_hist_body(tn_hbm, out_hbm, chunk0_v, chunk1_v, sum_v, sem0, sem1):
    wid = lax.axis_index("s") * _NSC + lax.axis_index("c")
    smp = wid // 2
    row0 = (wid % 2) * (_H // 2)
    zeros = jnp.zeros((16,), jnp.float32)
    bufs = (chunk0_v, chunk1_v)
    sems = (sem0, sem1)

    def _src(c):
        return tn_hbm.at[smp, pl.ds(row0 + c * _ROWS, _ROWS)]

    pltpu.make_async_copy(_src(0), chunk0_v, sem0).start()

    def zbody(i, carry):
        sum_v[pl.ds(i * 16, 16)] = zeros
        return carry

    lax.fori_loop(0, _BINS // 16, zbody, 0)

    def _process(buf):
        def ibody(r, icarry):
            vs = [buf[r, pl.ds(j * 16, 16)] for j in range(_W // 16)]
            bs = [
                jnp.minimum(
                    lax.convert_element_type(v * float(_BINS), jnp.int32),
                    _BINS - 1,
                )
                for v in vs
            ]
            for v, b in zip(vs, bs):
                plsc.addupdate_scatter(sum_v, [b], v)
            return icarry

        lax.fori_loop(0, _ROWS, ibody, 0)

    def cbody(g, carry):
        for b in range(2):
            c = 2 * g + b
            pltpu.make_async_copy(_src(c), bufs[b], sems[b]).wait()

            @pl.when(c + 1 < _NCHUNK)
            def _():
                pltpu.make_async_copy(
                    _src(c + 1), bufs[1 - b], sems[1 - b]
                ).start()

            _process(bufs[b])
        return carry

    lax.fori_loop(0, _NCHUNK // 2, cbody, 0)
    pltpu.sync_copy(sum_v, out_hbm.at[wid])


@functools.cache
def _make_sc_hist():
    mesh = plsc.VectorSubcoreMesh(
        core_axis_name="c",
        subcore_axis_name="s",
        num_cores=_NSC,
        num_subcores=_NSUB,
    )
    return pl.kernel(
        _sc_hist_body,
        out_type=jax.ShapeDtypeStruct((_NW, _BINS), jnp.float32),
        mesh=mesh,
        scratch_types=[
            pltpu.VMEM((_ROWS, _W), jnp.float32),
            pltpu.VMEM((_ROWS, _W), jnp.float32),
            pltpu.VMEM((_BINS,), jnp.float32),
            pltpu.SemaphoreType.DMA,
            pltpu.SemaphoreType.DMA,
        ],
        compiler_params=pltpu.CompilerParams(needs_layout_passes=False),
    )


def _fin_body(hist_ref, topm_ref):
    h = hist_ref[...].reshape(_N, 2, _BINS)
    hsum = h[:, 0] + h[:, 1]
    iota_b = lax.broadcasted_iota(jnp.int32, (_N, _BINS), 1)
    cnt_lo = hsum * (
        float(_BINS) / (lax.convert_element_type(iota_b, jnp.float32) + 1.0)
    )
    rc = cnt_lo
    k = 1
    while k < _BINS:
        rc = rc + jnp.concatenate(
            [rc[:, k:], jnp.zeros((_N, k), jnp.float32)], axis=1
        )
        k *= 2
    bstar = jnp.max(
        jnp.where(rc >= float(_M), iota_b, 0), axis=1, keepdims=True
    )
    t = lax.convert_element_type(bstar, jnp.float32) * (1.0 / float(_BINS))
    above = iota_b >= bstar
    c_above = jnp.sum(jnp.where(above, cnt_lo, 0.0), axis=1, keepdims=True)
    s_above = jnp.sum(jnp.where(above, hsum, 0.0), axis=1, keepdims=True)
    topm_ref[...] = s_above - t * c_above + float(_M) * t


def kernel(logits, targets):
    stats, tn = pl.pallas_call(
        _dense_body,
        grid=(_N,),
        in_specs=[
            pl.BlockSpec((1, _H, _W), lambda i: (i, 0, 0)),
            pl.BlockSpec((1, _H, _W), lambda i: (i, 0, 0)),
        ],
        out_specs=[
            pl.BlockSpec((1, 1, 4), lambda i: (i, 0, 0)),
            pl.BlockSpec((1, _H, _W), lambda i: (i, 0, 0)),
        ],
        out_shape=[
            jax.ShapeDtypeStruct((_N, 1, 4), jnp.float32),
            jax.ShapeDtypeStruct((_N, _H, _W), jnp.float32),
        ],
    )(logits, targets)

    hists = _make_sc_hist()(tn)

    topm = pl.pallas_call(
        _fin_body,
        out_shape=jax.ShapeDtypeStruct((_N, 1), jnp.float32),
    )(hists)

    s1 = stats[:, 0, 0]
    s2 = stats[:, 0, 1]
    s12 = stats[:, 0, 2]
    tm = topm[:, 0]
    score = 2.0 * (s12 + 1.0) / (s1 + 2.0 * s2 - s12 + tm + 1.0)
    return (1.0 - jnp.sum(score) / _N).astype(jnp.float32)


# speedup vs baseline: 1.8558x; 1.0213x over previous
import functools

import jax
import jax.numpy as jnp
from jax import lax
from jax.experimental import pallas as pl
from jax.experimental.pallas import tpu as pltpu
from jax.experimental.pallas import tpu_sc as plsc

_N = 16
_H = 512
_W = 512
_L = _H * _W
_M = int(0.1 * _L)

_NSC = 2
_NSUB = 16
_NW = _NSC * _NSUB
_HALF = _L * _N // _NW
_CHUNK = 16384
_NCHUNK = _HALF // _CHUNK
_BINS = 8192
_UNROLL = 8


def _dense_body(lg_ref, tg_ref, stats_ref, tn_ref):
    lg = lg_ref[0]
    tg = tg_ref[0]
    m1 = jax.nn.sigmoid(lg)
    tn = (1.0 - m1) * (1.0 - tg)
    tn_ref[0] = tn
    s1 = jnp.sum(m1)
    s2 = jnp.sum(tg)
    s12 = jnp.sum(m1 * tg)
    idx = lax.broadcasted_iota(jnp.int32, (1, 4), 1)
    stats_ref[0] = jnp.where(
        idx == 0, s1, jnp.where(idx == 1, s2, jnp.where(idx == 2, s12, 0.0))
    )


_ROWS = _CHUNK // _W
_NCHUNK_H = (_H // 4) // _ROWS


def _sc_hist_body(tn_hbm, out_hbm, chunk0_v, chunk1_v, sum_v, sem0, sem1):
    wid = lax.axis_index("s") * _NSC + lax.axis_index("c")
    smp = wid // 4
    row0 = (wid % 4) * (_H // 4)
    zeros = jnp.zeros((16,), jnp.float32)
    bufs = (chunk0_v, chunk1_v)
    sems = (sem0, sem1)

    def _src(c):
        return tn_hbm.at[smp, pl.ds(row0 + c * _ROWS, _ROWS)]

    pltpu.make_async_copy(_src(0), chunk0_v, sem0).start()

    def zbody(i, carry):
        sum_v[pl.ds(i * 16, 16)] = zeros
        return carry

    lax.fori_loop(0, _BINS // 16, zbody, 0)

    def _process(buf):
        def ibody(r, icarry):
            vs = [buf[r, pl.ds(j * 16, 16)] for j in range(_W // 16)]
            bs = [
                jnp.minimum(
                    lax.convert_element_type(v * float(_BINS), jnp.int32),
                    _BINS - 1,
                )
                for v in vs
            ]
            for v, b in zip(vs, bs):
                plsc.addupdate_scatter(sum_v, [b], v)
            return icarry

        lax.fori_loop(0, _ROWS, ibody, 0)

    def cbody(g, carry):
        for b in range(2):
            c = 2 * g + b
            pltpu.make_async_copy(_src(c), bufs[b], sems[b]).wait()

            @pl.when(c + 1 < _NCHUNK_H)
            def _():
                pltpu.make_async_copy(
                    _src(c + 1), bufs[1 - b], sems[1 - b]
                ).start()

            _process(bufs[b])
        return carry

    lax.fori_loop(0, _NCHUNK_H // 2, cbody, 0)
    pltpu.sync_copy(sum_v, out_hbm.at[wid])


@functools.cache
def _make_sc_hist():
    mesh = plsc.VectorSubcoreMesh(
        core_axis_name="c",
        subcore_axis_name="s",
        num_cores=_NSC,
        num_subcores=_NSUB,
    )
    return pl.kernel(
        _sc_hist_body,
        out_type=jax.ShapeDtypeStruct((_NW, _BINS), jnp.float32),
        name="sc_hist_half",
        mesh=mesh,
        scratch_types=[
            pltpu.VMEM((_ROWS, _W), jnp.float32),
            pltpu.VMEM((_ROWS, _W), jnp.float32),
            pltpu.VMEM((_BINS,), jnp.float32),
            pltpu.SemaphoreType.DMA,
            pltpu.SemaphoreType.DMA,
        ],
        compiler_params=pltpu.CompilerParams(needs_layout_passes=False),
    )


def _fin_body(hist0_ref, hist1_ref, topm_ref):
    h = jnp.concatenate(
        [
            hist0_ref[...].reshape(_N // 2, 4, _BINS),
            hist1_ref[...].reshape(_N // 2, 4, _BINS),
        ],
        axis=0,
    )
    hsum = h[:, 0] + h[:, 1] + h[:, 2] + h[:, 3]
    iota_b = lax.broadcasted_iota(jnp.int32, (_N, _BINS), 1)
    cnt_lo = hsum * (
        float(_BINS) / (lax.convert_element_type(iota_b, jnp.float32) + 1.0)
    )
    rc = cnt_lo
    k = 1
    while k < _BINS:
        rc = rc + jnp.concatenate(
            [rc[:, k:], jnp.zeros((_N, k), jnp.float32)], axis=1
        )
        k *= 2
    bstar = jnp.max(
        jnp.where(rc >= float(_M), iota_b, 0), axis=1, keepdims=True
    )
    t = lax.convert_element_type(bstar, jnp.float32) * (1.0 / float(_BINS))
    above = iota_b >= bstar
    c_above = jnp.sum(jnp.where(above, cnt_lo, 0.0), axis=1, keepdims=True)
    s_above = jnp.sum(jnp.where(above, hsum, 0.0), axis=1, keepdims=True)
    topm_ref[...] = s_above - t * c_above + float(_M) * t


def kernel(logits, targets):
    nh = _N // 2
    stats_h = []
    hists_h = []
    for h in range(2):
        stats1, tn1 = pl.pallas_call(
            _dense_body,
            grid=(nh,),
            in_specs=[
                pl.BlockSpec((1, _H, _W), lambda i, h=h: (i + h * nh, 0, 0)),
                pl.BlockSpec((1, _H, _W), lambda i, h=h: (i + h * nh, 0, 0)),
            ],
            out_specs=[
                pl.BlockSpec((1, 1, 4), lambda i: (i, 0, 0)),
                pl.BlockSpec((1, _H, _W), lambda i: (i, 0, 0)),
            ],
            out_shape=[
                jax.ShapeDtypeStruct((nh, 1, 4), jnp.float32),
                jax.ShapeDtypeStruct((nh, _H, _W), jnp.float32),
            ],
        )(logits, targets)
        stats_h.append(stats1)
        hists_h.append(_make_sc_hist()(tn1))

    stats = jnp.concatenate(stats_h, axis=0)
    topm = pl.pallas_call(
        _fin_body,
        out_shape=jax.ShapeDtypeStruct((_N, 1), jnp.float32),
    )(hists_h[0], hists_h[1])

    s1 = stats[:, 0, 0]
    s2 = stats[:, 0, 1]
    s12 = stats[:, 0, 2]
    tm = topm[:, 0]
    score = 2.0 * (s12 + 1.0) / (s1 + 2.0 * s2 - s12 + tm + 1.0)
    return (1.0 - jnp.sum(score) / _N).astype(jnp.float32)


# speedup vs baseline: 2.0877x; 1.1249x over previous
import functools

import jax
import jax.numpy as jnp
from jax import lax
from jax.experimental import pallas as pl
from jax.experimental.pallas import tpu as pltpu
from jax.experimental.pallas import tpu_sc as plsc

_N = 16
_H = 512
_W = 512
_L = _H * _W
_M = int(0.1 * _L)

_NSC = 2
_NSUB = 16
_NW = _NSC * _NSUB
_HALF = _L * _N // _NW
_CHUNK = 16384
_NCHUNK = _HALF // _CHUNK
_BINS = 2048
_UNROLL = 8


def _dense_body(lg_ref, tg_ref, stats_ref, tn_ref):
    lg = lg_ref[0]
    tg = tg_ref[0]
    m1 = jax.nn.sigmoid(lg)
    tn = (1.0 - m1) * (1.0 - tg)
    tn_ref[0] = tn
    s1 = jnp.sum(m1)
    s2 = jnp.sum(tg)
    s12 = jnp.sum(m1 * tg)
    idx = lax.broadcasted_iota(jnp.int32, (1, 4), 1)
    stats_ref[0] = jnp.where(
        idx == 0, s1, jnp.where(idx == 1, s2, jnp.where(idx == 2, s12, 0.0))
    )


_ROWS = _CHUNK // _W
_NCHUNK_H = (_H // 4) // _ROWS


def _sc_hist_body(tn_hbm, out_hbm, chunk0_v, chunk1_v, sum_v, sem0, sem1):
    wid = lax.axis_index("s") * _NSC + lax.axis_index("c")
    smp = wid // 4
    row0 = (wid % 4) * (_H // 4)
    zeros = jnp.zeros((16,), jnp.float32)
    bufs = (chunk0_v, chunk1_v)
    sems = (sem0, sem1)

    def _src(c):
        return tn_hbm.at[smp, pl.ds(row0 + c * _ROWS, _ROWS)]

    pltpu.make_async_copy(_src(0), chunk0_v, sem0).start()

    def zbody(i, carry):
        sum_v[pl.ds(i * 16, 16)] = zeros
        return carry

    lax.fori_loop(0, _BINS // 16, zbody, 0)

    def _process(buf):
        def ibody(r, icarry):
            vs = [buf[r, pl.ds(j * 16, 16)] for j in range(_W // 16)]
            bs = [
                jnp.minimum(
                    lax.convert_element_type(v * float(_BINS), jnp.int32),
                    _BINS - 1,
                )
                for v in vs
            ]
            for v, b in zip(vs, bs):
                plsc.addupdate_scatter(sum_v, [b], v)
            return icarry

        lax.fori_loop(0, _ROWS, ibody, 0)

    def cbody(g, carry):
        for b in range(2):
            c = 2 * g + b
            pltpu.make_async_copy(_src(c), bufs[b], sems[b]).wait()

            @pl.when(c + 1 < _NCHUNK_H)
            def _():
                pltpu.make_async_copy(
                    _src(c + 1), bufs[1 - b], sems[1 - b]
                ).start()

            _process(bufs[b])
        return carry

    lax.fori_loop(0, _NCHUNK_H // 2, cbody, 0)
    pltpu.sync_copy(sum_v, out_hbm.at[wid])


@functools.cache
def _make_sc_hist():
    mesh = plsc.VectorSubcoreMesh(
        core_axis_name="c",
        subcore_axis_name="s",
        num_cores=_NSC,
        num_subcores=_NSUB,
    )
    return pl.kernel(
        _sc_hist_body,
        out_type=jax.ShapeDtypeStruct((_NW, _BINS), jnp.float32),
        name="sc_hist_half",
        mesh=mesh,
        scratch_types=[
            pltpu.VMEM((_ROWS, _W), jnp.float32),
            pltpu.VMEM((_ROWS, _W), jnp.float32),
            pltpu.VMEM((_BINS,), jnp.float32),
            pltpu.SemaphoreType.DMA,
            pltpu.SemaphoreType.DMA,
        ],
        compiler_params=pltpu.CompilerParams(needs_layout_passes=False),
    )


def _fin_body(hist0_ref, hist1_ref, topm_ref):
    h = jnp.concatenate(
        [
            hist0_ref[...].reshape(_N // 2, 4, _BINS),
            hist1_ref[...].reshape(_N // 2, 4, _BINS),
        ],
        axis=0,
    )
    hsum = h[:, 0] + h[:, 1] + h[:, 2] + h[:, 3]
    iota_b = lax.broadcasted_iota(jnp.int32, (_N, _BINS), 1)
    cnt_lo = hsum * (
        float(_BINS) / (lax.convert_element_type(iota_b, jnp.float32) + 1.0)
    )
    rc = cnt_lo
    k = 1
    while k < _BINS:
        rc = rc + jnp.concatenate(
            [rc[:, k:], jnp.zeros((_N, k), jnp.float32)], axis=1
        )
        k *= 2
    bstar = jnp.max(
        jnp.where(rc >= float(_M), iota_b, 0), axis=1, keepdims=True
    )
    t = lax.convert_element_type(bstar, jnp.float32) * (1.0 / float(_BINS))
    above = iota_b >= bstar
    c_above = jnp.sum(jnp.where(above, cnt_lo, 0.0), axis=1, keepdims=True)
    s_above = jnp.sum(jnp.where(above, hsum, 0.0), axis=1, keepdims=True)
    topm_ref[...] = s_above - t * c_above + float(_M) * t


def kernel(logits, targets):
    nh = _N // 2
    stats_h = []
    hists_h = []
    for h in range(2):
        stats1, tn1 = pl.pallas_call(
            _dense_body,
            grid=(nh,),
            in_specs=[
                pl.BlockSpec((1, _H, _W), lambda i, h=h: (i + h * nh, 0, 0)),
                pl.BlockSpec((1, _H, _W), lambda i, h=h: (i + h * nh, 0, 0)),
            ],
            out_specs=[
                pl.BlockSpec((1, 1, 4), lambda i: (i, 0, 0)),
                pl.BlockSpec((1, _H, _W), lambda i: (i, 0, 0)),
            ],
            out_shape=[
                jax.ShapeDtypeStruct((nh, 1, 4), jnp.float32),
                jax.ShapeDtypeStruct((nh, _H, _W), jnp.float32),
            ],
        )(logits, targets)
        stats_h.append(stats1)
        hists_h.append(_make_sc_hist()(tn1))

    stats = jnp.concatenate(stats_h, axis=0)
    topm = pl.pallas_call(
        _fin_body,
        out_shape=jax.ShapeDtypeStruct((_N, 1), jnp.float32),
    )(hists_h[0], hists_h[1])

    s1 = stats[:, 0, 0]
    s2 = stats[:, 0, 1]
    s12 = stats[:, 0, 2]
    tm = topm[:, 0]
    score = 2.0 * (s12 + 1.0) / (s1 + 2.0 * s2 - s12 + tm + 1.0)
    return (1.0 - jnp.sum(score) / _N).astype(jnp.float32)
